# baseline (speedup 1.0000x reference)
import functools

import jax
import jax.numpy as jnp
from jax import lax
from jax.experimental import pallas as pl
from jax.experimental.pallas import tpu as pltpu
from jax.experimental.pallas import tpu_sc as plsc

N_USER = 100000
M_ITEM = 100000
DIM = 64
BATCH = 4096

_L = 16
_NC = 2
_NS = 16
_NW = _NC * _NS
_BPW = BATCH // _NW
_VPR = DIM // _L
_PAIR = 2 * DIM


def _lane_shuffle(x, idx):
    return lax.gather(
        x,
        idx.reshape(_L, 1),
        lax.GatherDimensionNumbers(
            offset_dims=(), collapsed_slice_dims=(0,), start_index_map=(0,)),
        slice_sizes=(1,),
        mode=lax.GatherScatterMode.PROMISE_IN_BOUNDS,
    )


def _sc_scores_kernel(user_hbm, item_hbm, uid_hbm, iid_hbm, out_hbm,
                      uidx_v, iidx_v, upair_v, ipair_v, urows_v, irows_v,
                      out_v, sem_u, sem_i):
    wid = lax.axis_index("s") * _NC + lax.axis_index("c")
    base = wid * _BPW

    pltpu.sync_copy(uid_hbm.at[pl.ds(base, _BPW)], uidx_v)
    pltpu.sync_copy(iid_hbm.at[pl.ds(base, _BPW)], iidx_v)

    def halve(g, carry):
        upair_v[pl.ds(g * _L, _L)] = uidx_v[pl.ds(g * _L, _L)] >> 1
        ipair_v[pl.ds(g * _L, _L)] = iidx_v[pl.ds(g * _L, _L)] >> 1
        return carry

    lax.fori_loop(0, _BPW // _L, halve, 0, unroll=True)

    cp_u = pltpu.async_copy(user_hbm.at[upair_v], urows_v, sem_u)
    cp_i = pltpu.async_copy(item_hbm.at[ipair_v], irows_v, sem_i)
    cp_u.wait()
    cp_i.wait()

    lane = lax.iota(jnp.int32, _L)
    perms = [lane ^ k for k in (1, 2, 4, 8)]

    def group_body(g, carry):
        uids = uidx_v[pl.ds(g * _L, _L)]
        iids = iidx_v[pl.ds(g * _L, _L)]
        uoff = (uids & 1) * DIM
        ioff = (iids & 1) * DIM
        out_vec = jnp.zeros((_L,), jnp.float32)
        for r in range(_L):
            b = g * _L + r
            uo = uoff[r]
            io = ioff[r]
            acc = (urows_v[b, pl.ds(uo, _L)] * irows_v[b, pl.ds(io, _L)])
            for j in range(1, _VPR):
                acc = acc + (urows_v[b, pl.ds(uo + j * _L, _L)]
                             * irows_v[b, pl.ds(io + j * _L, _L)])
            for p in perms:
                acc = acc + _lane_shuffle(acc, p)
            out_vec = jnp.where(lane == r, acc, out_vec)
        out_v[pl.ds(g * _L, _L)] = out_vec
        return carry

    lax.fori_loop(0, _BPW // _L, group_body, 0)

    pltpu.sync_copy(out_v, out_hbm.at[pl.ds(base, _BPW)])


@jax.jit
def kernel(user_table, item_table, user_ids, item_ids):
    mesh = plsc.VectorSubcoreMesh(core_axis_name="c", subcore_axis_name="s")
    run = functools.partial(
        pl.kernel,
        mesh=mesh,
        out_type=jax.ShapeDtypeStruct((BATCH,), jnp.float32),
        scratch_types=[
            pltpu.VMEM((_BPW,), jnp.int32),
            pltpu.VMEM((_BPW,), jnp.int32),
            pltpu.VMEM((_BPW,), jnp.int32),
            pltpu.VMEM((_BPW,), jnp.int32),
            pltpu.VMEM((_BPW, _PAIR), jnp.float32),
            pltpu.VMEM((_BPW, _PAIR), jnp.float32),
            pltpu.VMEM((_BPW,), jnp.float32),
            pltpu.SemaphoreType.DMA,
            pltpu.SemaphoreType.DMA,
        ],
        compiler_params=pltpu.CompilerParams(use_tc_tiling_on_sc=True),
    )(_sc_scores_kernel)
    return run(user_table.reshape(N_USER // 2, _PAIR),
               item_table.reshape(M_ITEM // 2, _PAIR),
               user_ids.astype(jnp.int32), item_ids.astype(jnp.int32))


# speedup vs baseline: 1.2393x; 1.2393x over previous
import functools

import jax
import jax.numpy as jnp
from jax import lax
from jax.experimental import pallas as pl
from jax.experimental.pallas import tpu as pltpu
from jax.experimental.pallas import tpu_sc as plsc

N_USER = 100000
M_ITEM = 100000
DIM = 64
BATCH = 4096

_L = 16
_NC = 2
_NS = 16
_NW = _NC * _NS
_BPW = BATCH // _NW
_VPR = DIM // _L
_W = 32
_NWAVE = _BPW // _W


def _lane_shuffle(x, idx):
    return lax.gather(
        x,
        idx.reshape(_L, 1),
        lax.GatherDimensionNumbers(
            offset_dims=(), collapsed_slice_dims=(0,), start_index_map=(0,)),
        slice_sizes=(1,),
        mode=lax.GatherScatterMode.PROMISE_IN_BOUNDS,
    )


def _sc_scores_kernel(user_hbm, item_hbm, uid_hbm, iid_hbm, out_hbm,
                      uidx_v, iidx_v, ublk_v, iblk_v, out_v, sem_u, sem_i):
    wid = lax.axis_index("s") * _NC + lax.axis_index("c")
    base = wid * _BPW

    pltpu.sync_copy(uid_hbm.at[pl.ds(base, _BPW)], uidx_v)
    pltpu.sync_copy(iid_hbm.at[pl.ds(base, _BPW)], iidx_v)

    lane = lax.iota(jnp.int32, _L)
    perms = [lane ^ k for k in (1, 2, 4, 8)]

    for w in range(_NWAVE):
        copies = []
        idvecs = []
        for g in range(_W // _L):
            uids = uidx_v[pl.ds((w * _W + g * _L), _L)]
            iids = iidx_v[pl.ds((w * _W + g * _L), _L)]
            idvecs.append((uids, iids))
            for r in range(_L):
                s = g * _L + r
                ublk = pl.multiple_of((uids[r] >> 3) * 8, 8)
                iblk = pl.multiple_of((iids[r] >> 3) * 8, 8)
                copies.append(pltpu.async_copy(
                    user_hbm.at[pl.ds(ublk, 8), :], ublk_v.at[s], sem_u))
                copies.append(pltpu.async_copy(
                    item_hbm.at[pl.ds(iblk, 8), :], iblk_v.at[s], sem_i))
        for cp in copies:
            cp.wait()

        for g in range(_W // _L):
            uids, iids = idvecs[g]
            urow = uids & 7
            irow = iids & 7
            out_vec = jnp.zeros((_L,), jnp.float32)
            for r in range(_L):
                s = g * _L + r
                uo = urow[r]
                io = irow[r]
                acc = (ublk_v[s, uo, pl.ds(0, _L)]
                       * iblk_v[s, io, pl.ds(0, _L)])
                for j in range(1, _VPR):
                    acc = acc + (ublk_v[s, uo, pl.ds(j * _L, _L)]
                                 * iblk_v[s, io, pl.ds(j * _L, _L)])
                for p in perms:
                    acc = acc + _lane_shuffle(acc, p)
                out_vec = jnp.where(lane == r, acc, out_vec)
            out_v[pl.ds(w * _W + g * _L, _L)] = out_vec

    pltpu.sync_copy(out_v, out_hbm.at[pl.ds(base, _BPW)])


@jax.jit
def kernel(user_table, item_table, user_ids, item_ids):
    mesh = plsc.VectorSubcoreMesh(core_axis_name="c", subcore_axis_name="s")
    run = functools.partial(
        pl.kernel,
        mesh=mesh,
        out_type=jax.ShapeDtypeStruct((BATCH,), jnp.float32),
        scratch_types=[
            pltpu.VMEM((_BPW,), jnp.int32),
            pltpu.VMEM((_BPW,), jnp.int32),
            pltpu.VMEM((_W, 8, DIM), jnp.float32),
            pltpu.VMEM((_W, 8, DIM), jnp.float32),
            pltpu.VMEM((_BPW,), jnp.float32),
            pltpu.SemaphoreType.DMA,
            pltpu.SemaphoreType.DMA,
        ],
        compiler_params=pltpu.CompilerParams(use_tc_tiling_on_sc=True),
    )(_sc_scores_kernel)
    return run(user_table, item_table,
               user_ids.astype(jnp.int32), item_ids.astype(jnp.int32))
